# Initial kernel scaffold; baseline (speedup 1.0000x reference)
import jax
import jax.numpy as jnp
from jax.experimental import pallas as pl


def kernel(x, edge_index, W0, b0, W1, b1):
    raise NotImplementedError("write your pallas kernel here")


# speedup vs baseline: 3.5420x; 3.5420x over previous
import functools

import jax
import jax.numpy as jnp
from jax import lax
from jax.experimental import pallas as pl
from jax.experimental.pallas import tpu as pltpu
from jax.experimental.pallas import tpu_sc as plsc

N = 50000
D = 64
E = 800000
QD = 16

NC, NS = 2, 16
NW = NC * NS
NPAD = 50176
Q8 = NPAD // 8
Q16 = NPAD // 16
ZR = NPAD // NS
EPAD = 819200
ER = EPAD // 128
RT = ER // NW
RTA = ER // NS
CR = 8
NCHUNK = RT // CR
NCHUNKA = RTA // CR
DEGW = 8

_sc_params = pltpu.CompilerParams(use_tc_tiling_on_sc=False)


def _sc_agg_body(hq, srcb, dst2, zeros, aggq, *rest):
    idx_s = rest[:CR]
    idx_d, rows, sem, acc = rest[CR:]
    c = lax.axis_index("c")
    s = lax.axis_index("s")
    z0 = s * ZR
    base = s * RTA

    for p in range(2):
        q = c * 2 + p
        pltpu.sync_copy(zeros, acc.at[pl.ds(z0, ZR)])
        plsc.subcore_barrier()

        def chunk(i, carry):
            e0 = q * EPAD + (base + i * CR) * 128
            for j in range(CR):
                pltpu.sync_copy(
                    srcb.at[pl.ds(e0 + j * 128, 128)], idx_s[j]
                )
            pltpu.sync_copy(dst2.at[pl.ds(base + i * CR, CR)], idx_d)
            cps = [
                pltpu.async_copy(
                    hq.at[idx_s[j]], rows.at[pl.ds(j * 128, 128)], sem
                )
                for j in range(CR)
            ]
            for cp in cps:
                cp.wait()
            for j in range(CR):
                pltpu.sync_copy(
                    rows.at[pl.ds(j * 128, 128)], acc.at[idx_d.at[j]], add=True
                )
            return carry

        lax.fori_loop(0, NCHUNKA, chunk, 0)
        plsc.subcore_barrier()
        pltpu.sync_copy(acc.at[pl.ds(z0, ZR)], aggq.at[pl.ds(q * NPAD + z0, ZR)])
        plsc.subcore_barrier()


def _sc_deg_body(dst2, onesd, zerosd, deg2, idx_d, ones_v, acc):
    c = lax.axis_index("c")
    s = lax.axis_index("s")
    wid = s * NC + c
    z0 = s * ZR
    pltpu.sync_copy(zerosd, acc.at[pl.ds(z0, ZR)])
    pltpu.sync_copy(onesd, ones_v)
    plsc.subcore_barrier()

    base = wid * RT

    def chunk(i, carry):
        r0 = base + i * CR
        pltpu.sync_copy(dst2.at[pl.ds(r0, CR)], idx_d)
        for j in range(CR):
            pltpu.sync_copy(ones_v, acc.at[idx_d.at[j]], add=True)
        return carry

    lax.fori_loop(0, NCHUNK, chunk, 0)
    plsc.subcore_barrier()
    pltpu.sync_copy(acc.at[pl.ds(z0, ZR)], deg2.at[pl.ds(c * NPAD + z0, ZR)])


@functools.cache
def _build_sc_kernels():
    mesh = plsc.VectorSubcoreMesh(
        core_axis_name="c", subcore_axis_name="s", num_cores=NC, num_subcores=NS
    )
    sc_agg = pl.kernel(
        _sc_agg_body,
        out_type=jax.ShapeDtypeStruct((4 * NPAD, QD), jnp.float32),
        mesh=mesh,
        compiler_params=_sc_params,
        scratch_types=[pltpu.VMEM((128,), jnp.int32) for _ in range(CR)]
        + [
            pltpu.VMEM((CR, 128), jnp.int32),
            pltpu.VMEM((CR * 128, QD), jnp.float32),
            pltpu.SemaphoreType.DMA,
            pltpu.VMEM_SHARED((NPAD, QD), jnp.float32),
        ],
    )
    sc_deg = pl.kernel(
        _sc_deg_body,
        out_type=jax.ShapeDtypeStruct((2 * NPAD, DEGW), jnp.float32),
        mesh=mesh,
        compiler_params=_sc_params,
        scratch_types=[
            pltpu.VMEM((CR, 128), jnp.int32),
            pltpu.VMEM((128, DEGW), jnp.float32),
            pltpu.VMEM_SHARED((NPAD, DEGW), jnp.float32),
        ],
    )
    return sc_agg, sc_deg


B8 = 128
BD = B8 // 2
GRID = Q8 // B8


def _tc_body(h, a, d0, d1, ws, b8, pmat, smat, o_ref, *, last):
    f32 = jnp.float32
    d = d0[0] + d1[0]
    drep = jnp.dot(pmat[...], d, preferred_element_type=f32)
    rid = lax.broadcasted_iota(jnp.int32, (B8, 1), 0)
    dsel = jnp.where(
        rid % 2 == 0,
        jnp.dot(drep, smat[0], preferred_element_type=f32),
        jnp.dot(drep, smat[1], preferred_element_type=f32),
    )
    inv = 1.0 / jnp.maximum(dsel, 1.0)
    z = b8[...]
    for q in range(4):
        z = z + jnp.dot(h[q], ws[q], preferred_element_type=f32)
    for q in range(4):
        z = z + jnp.dot(a[q] * inv, ws[4 + q], preferred_element_type=f32)
    if last:
        parts = []
        for g in range(8):
            zg = z[:, 64 * g : 64 * g + 64]
            n2 = jnp.sum(zg * zg, axis=1, keepdims=True)
            nrm = jnp.maximum(jnp.sqrt(n2), 1e-12)
            parts.append(zg / nrm)
        o_ref[...] = jnp.concatenate(parts, axis=1)
    else:
        z = jnp.maximum(z, 0.0)
        for q in range(4):
            o_ref[q] = jnp.concatenate(
                [z[:, 64 * g + QD * q : 64 * g + QD * q + QD] for g in range(8)],
                axis=1,
            )


def _tc_layer(hT, aggT, degT, WS, b8, pmat, smat, last):
    out_shape = (
        jax.ShapeDtypeStruct((Q8, 8 * D), jnp.float32)
        if last
        else jax.ShapeDtypeStruct((4, Q8, 128), jnp.float32)
    )
    out_spec = (
        pl.BlockSpec((B8, 8 * D), lambda i: (i, 0))
        if last
        else pl.BlockSpec((4, B8, 128), lambda i: (0, i, 0))
    )
    return pl.pallas_call(
        functools.partial(_tc_body, last=last),
        grid=(GRID,),
        in_specs=[
            pl.BlockSpec((4, B8, 128), lambda i: (0, i, 0)),
            pl.BlockSpec((4, B8, 128), lambda i: (0, i, 0)),
            pl.BlockSpec((1, BD, 128), lambda i: (0, i, 0)),
            pl.BlockSpec((1, BD, 128), lambda i: (1, i, 0)),
            pl.BlockSpec((8, 2 * D, 8 * D), lambda i: (0, 0, 0)),
            pl.BlockSpec((1, 8 * D), lambda i: (0, 0)),
            pl.BlockSpec((B8, BD), lambda i: (0, 0)),
            pl.BlockSpec((2, 128, 128), lambda i: (0, 0, 0)),
        ],
        out_specs=out_spec,
        out_shape=out_shape,
    )(hT, aggT, degT, degT, WS, b8, pmat, smat)


def _block_diag8(w):
    z = jnp.zeros((QD, D), jnp.float32)
    rows = []
    for i in range(8):
        cols = [w if j == i else z for j in range(8)]
        rows.append(jnp.concatenate(cols, axis=1))
    return jnp.concatenate(rows, axis=0)


def _pack_weights(W, b):
    WS = jnp.stack(
        [_block_diag8(W[QD * k : QD * k + QD]) for k in range(8)]
    )
    b8 = jnp.tile(b.reshape(1, D), (1, 8))
    return WS, b8


def kernel(x, edge_index, W0, b0, W1, b1):
    src = edge_index[0]
    dst = edge_index[1]
    xp = jnp.pad(x, ((0, NPAD - N), (0, 0)))
    hT0 = jnp.stack(
        [xp[:, QD * q : QD * q + QD].reshape(Q8, 128) for q in range(4)]
    )

    src_p = jnp.concatenate([src, jnp.zeros((EPAD - E,), jnp.int32)])
    dst_p = jnp.concatenate([dst, jnp.full((EPAD - E,), N, jnp.int32)])
    srcb = jnp.concatenate([src_p + q * NPAD for q in range(4)])
    dst2 = dst_p.reshape(ER, 128)

    zeros = jnp.zeros((ZR, QD), jnp.float32)
    zerosd = jnp.zeros((ZR, DEGW), jnp.float32)
    onesd = jnp.ones((128, DEGW), jnp.float32)
    pmat = (
        jnp.arange(B8)[:, None] // 2 == jnp.arange(BD)[None, :]
    ).astype(jnp.float32)
    lane = jnp.arange(128)
    smat = jnp.stack(
        [
            (jnp.arange(128)[:, None] == 64 * m + 8 * (lane[None, :] // 16)).astype(
                jnp.float32
            )
            for m in range(2)
        ]
    )
    WS0, b80 = _pack_weights(W0, b0)
    WS1, b81 = _pack_weights(W1, b1)

    sc_agg, sc_deg = _build_sc_kernels()
    degT = sc_deg(dst2, onesd, zerosd).reshape(2, Q16, 128)

    agg0 = sc_agg(hT0.reshape(4 * NPAD, QD), srcb, dst2, zeros)
    h1T = _tc_layer(
        hT0, agg0.reshape(4, Q8, 128), degT, WS0, b80, pmat, smat, False
    )

    agg1 = sc_agg(h1T.reshape(4 * NPAD, QD), srcb, dst2, zeros)
    outp = _tc_layer(
        h1T, agg1.reshape(4, Q8, 128), degT, WS1, b81, pmat, smat, True
    )

    return outp.reshape(NPAD, D)[:N]


# speedup vs baseline: 5.7838x; 1.6329x over previous
import functools

import jax
import jax.numpy as jnp
from jax import lax
from jax.experimental import pallas as pl
from jax.experimental.pallas import tpu as pltpu
from jax.experimental.pallas import tpu_sc as plsc

N = 50000
D = 64
E = 800000
QD = 16

NC, NS = 2, 16
NW = NC * NS
NPAD = 50176
Q8 = NPAD // 8
Q16 = NPAD // 16
ZR = NPAD // NS
EPAD = 819200
ER = EPAD // 128
RT = ER // NW
CR = 8
NCHUNK = RT // CR
EPG = EPAD // NS
CB = 1600
NPAIR = EPG // (2 * CB)
DEGW = 8

_sc_params = pltpu.CompilerParams(use_tc_tiling_on_sc=False)


def _sc_agg_body(hq, srcb, dst1, zeros, aggq, *rest):
    idx_s = rest[0:2]
    idx_d = rest[2:4]
    rows = rest[4:6]
    semg = rest[6:8]
    sems = rest[8:10]
    acc = rest[10]
    c = lax.axis_index("c")
    s = lax.axis_index("s")
    z0 = s * ZR
    base_e = s * EPG

    def drain(b):
        pltpu.make_async_copy(rows[b], acc.at[idx_d[b]], sems[b]).wait()

    for p in range(2):
        q = c * 2 + p
        pltpu.sync_copy(zeros, acc.at[pl.ds(z0, ZR)])
        plsc.subcore_barrier()

        def pair(i2, carry):
            for b in range(2):
                e0 = base_e + (i2 * 2 + b) * CB

                @pl.when(i2 > 0)
                def _():
                    drain(b)

                pltpu.sync_copy(srcb.at[pl.ds(q * EPAD + e0, CB)], idx_s[b])
                pltpu.sync_copy(dst1.at[pl.ds(e0, CB)], idx_d[b])
                pltpu.async_copy(hq.at[idx_s[b]], rows[b], semg[b]).wait()
                pltpu.async_copy(rows[b], acc.at[idx_d[b]], sems[b], add=True)
            return carry

        lax.fori_loop(0, NPAIR, pair, 0)
        for b in range(2):
            drain(b)
        plsc.subcore_barrier()
        pltpu.sync_copy(acc.at[pl.ds(z0, ZR)], aggq.at[pl.ds(q * NPAD + z0, ZR)])
        plsc.subcore_barrier()


def _sc_deg_body(dst2, onesd, zerosd, deg2, idx_d, ones_v, acc):
    c = lax.axis_index("c")
    s = lax.axis_index("s")
    wid = s * NC + c
    z0 = s * ZR
    pltpu.sync_copy(zerosd, acc.at[pl.ds(z0, ZR)])
    pltpu.sync_copy(onesd, ones_v)
    plsc.subcore_barrier()

    base = wid * RT

    def chunk(i, carry):
        r0 = base + i * CR
        pltpu.sync_copy(dst2.at[pl.ds(r0, CR)], idx_d)
        for j in range(CR):
            pltpu.sync_copy(ones_v, acc.at[idx_d.at[j]], add=True)
        return carry

    lax.fori_loop(0, NCHUNK, chunk, 0)
    plsc.subcore_barrier()
    pltpu.sync_copy(acc.at[pl.ds(z0, ZR)], deg2.at[pl.ds(c * NPAD + z0, ZR)])


@functools.cache
def _build_sc_kernels():
    mesh = plsc.VectorSubcoreMesh(
        core_axis_name="c", subcore_axis_name="s", num_cores=NC, num_subcores=NS
    )
    sc_agg = pl.kernel(
        _sc_agg_body,
        out_type=jax.ShapeDtypeStruct((4 * NPAD, QD), jnp.float32),
        mesh=mesh,
        compiler_params=_sc_params,
        scratch_types=[pltpu.VMEM((CB,), jnp.int32) for _ in range(2)]
        + [pltpu.VMEM((CB,), jnp.int32) for _ in range(2)]
        + [pltpu.VMEM((CB, QD), jnp.float32) for _ in range(2)]
        + [pltpu.SemaphoreType.DMA for _ in range(2)]
        + [pltpu.SemaphoreType.DMA for _ in range(2)]
        + [pltpu.VMEM_SHARED((NPAD, QD), jnp.float32)],
    )
    sc_deg = pl.kernel(
        _sc_deg_body,
        out_type=jax.ShapeDtypeStruct((2 * NPAD, DEGW), jnp.float32),
        mesh=mesh,
        compiler_params=_sc_params,
        scratch_types=[
            pltpu.VMEM((CR, 128), jnp.int32),
            pltpu.VMEM((128, DEGW), jnp.float32),
            pltpu.VMEM_SHARED((NPAD, DEGW), jnp.float32),
        ],
    )
    return sc_agg, sc_deg


B8 = 128
BD = B8 // 2
GRID = Q8 // B8


def _tc_body(h, a, d0, d1, ws, b8, pmat, smat, o_ref, *, last):
    f32 = jnp.float32
    d = d0[0] + d1[0]
    drep = jnp.dot(pmat[...], d, preferred_element_type=f32)
    rid = lax.broadcasted_iota(jnp.int32, (B8, 1), 0)
    dsel = jnp.where(
        rid % 2 == 0,
        jnp.dot(drep, smat[0], preferred_element_type=f32),
        jnp.dot(drep, smat[1], preferred_element_type=f32),
    )
    inv = 1.0 / jnp.maximum(dsel, 1.0)
    z = b8[...]
    for q in range(4):
        z = z + jnp.dot(h[q], ws[q], preferred_element_type=f32)
    for q in range(4):
        z = z + jnp.dot(a[q] * inv, ws[4 + q], preferred_element_type=f32)
    if last:
        parts = []
        for g in range(8):
            zg = z[:, 64 * g : 64 * g + 64]
            n2 = jnp.sum(zg * zg, axis=1, keepdims=True)
            nrm = jnp.maximum(jnp.sqrt(n2), 1e-12)
            parts.append(zg / nrm)
        o_ref[...] = jnp.concatenate(parts, axis=1)
    else:
        z = jnp.maximum(z, 0.0)
        for q in range(4):
            o_ref[q] = jnp.concatenate(
                [z[:, 64 * g + QD * q : 64 * g + QD * q + QD] for g in range(8)],
                axis=1,
            )


def _tc_layer(hT, aggT, degT, WS, b8, pmat, smat, last):
    out_shape = (
        jax.ShapeDtypeStruct((Q8, 8 * D), jnp.float32)
        if last
        else jax.ShapeDtypeStruct((4, Q8, 128), jnp.float32)
    )
    out_spec = (
        pl.BlockSpec((B8, 8 * D), lambda i: (i, 0))
        if last
        else pl.BlockSpec((4, B8, 128), lambda i: (0, i, 0))
    )
    return pl.pallas_call(
        functools.partial(_tc_body, last=last),
        grid=(GRID,),
        in_specs=[
            pl.BlockSpec((4, B8, 128), lambda i: (0, i, 0)),
            pl.BlockSpec((4, B8, 128), lambda i: (0, i, 0)),
            pl.BlockSpec((1, BD, 128), lambda i: (0, i, 0)),
            pl.BlockSpec((1, BD, 128), lambda i: (1, i, 0)),
            pl.BlockSpec((8, 2 * D, 8 * D), lambda i: (0, 0, 0)),
            pl.BlockSpec((1, 8 * D), lambda i: (0, 0)),
            pl.BlockSpec((B8, BD), lambda i: (0, 0)),
            pl.BlockSpec((2, 128, 128), lambda i: (0, 0, 0)),
        ],
        out_specs=out_spec,
        out_shape=out_shape,
    )(hT, aggT, degT, degT, WS, b8, pmat, smat)


def _block_diag8(w):
    z = jnp.zeros((QD, D), jnp.float32)
    rows = []
    for i in range(8):
        cols = [w if j == i else z for j in range(8)]
        rows.append(jnp.concatenate(cols, axis=1))
    return jnp.concatenate(rows, axis=0)


def _pack_weights(W, b):
    WS = jnp.stack(
        [_block_diag8(W[QD * k : QD * k + QD]) for k in range(8)]
    )
    b8 = jnp.tile(b.reshape(1, D), (1, 8))
    return WS, b8


def kernel(x, edge_index, W0, b0, W1, b1):
    src = edge_index[0]
    dst = edge_index[1]
    xp = jnp.pad(x, ((0, NPAD - N), (0, 0)))
    hT0 = jnp.stack(
        [xp[:, QD * q : QD * q + QD].reshape(Q8, 128) for q in range(4)]
    )

    src_p = jnp.concatenate([src, jnp.zeros((EPAD - E,), jnp.int32)])
    dst_p = jnp.concatenate([dst, jnp.full((EPAD - E,), N, jnp.int32)])
    srcb = jnp.concatenate([src_p + q * NPAD for q in range(4)])
    dst2 = dst_p.reshape(ER, 128)

    zeros = jnp.zeros((ZR, QD), jnp.float32)
    zerosd = jnp.zeros((ZR, DEGW), jnp.float32)
    onesd = jnp.ones((128, DEGW), jnp.float32)
    pmat = (
        jnp.arange(B8)[:, None] // 2 == jnp.arange(BD)[None, :]
    ).astype(jnp.float32)
    lane = jnp.arange(128)
    smat = jnp.stack(
        [
            (jnp.arange(128)[:, None] == 64 * m + 8 * (lane[None, :] // 16)).astype(
                jnp.float32
            )
            for m in range(2)
        ]
    )
    WS0, b80 = _pack_weights(W0, b0)
    WS1, b81 = _pack_weights(W1, b1)

    sc_agg, sc_deg = _build_sc_kernels()
    degT = sc_deg(dst2, onesd, zerosd).reshape(2, Q16, 128)

    agg0 = sc_agg(hT0.reshape(4 * NPAD, QD), srcb, dst_p, zeros)
    h1T = _tc_layer(
        hT0, agg0.reshape(4, Q8, 128), degT, WS0, b80, pmat, smat, False
    )

    agg1 = sc_agg(h1T.reshape(4 * NPAD, QD), srcb, dst_p, zeros)
    outp = _tc_layer(
        h1T, agg1.reshape(4, Q8, 128), degT, WS1, b81, pmat, smat, True
    )

    return outp.reshape(NPAD, D)[:N]


# speedup vs baseline: 6.3203x; 1.0928x over previous
import functools

import jax
import jax.numpy as jnp
from jax import lax
from jax.experimental import pallas as pl
from jax.experimental.pallas import tpu as pltpu
from jax.experimental.pallas import tpu_sc as plsc

N = 50000
D = 64
E = 800000
QD = 16

NC, NS = 2, 16
NW = NC * NS
NPAD = 50176
Q8 = NPAD // 8
Q16 = NPAD // 16
ZR = NPAD // NS
EPAD = 819200
ER = EPAD // 128
RT = ER // NW
CR = 8
NCHUNK = RT // CR
EPG = EPAD // NS
CB = 1600
NPAIR = EPG // (2 * CB)
DEGW = 8

_sc_params = pltpu.CompilerParams(use_tc_tiling_on_sc=False)


def _sc_agg_body(hq, srcb, dst1, zeros, aggq, *rest):
    idx_s = rest[0:2]
    idx_d = rest[2:4]
    rows = rest[4:6]
    semg = rest[6:8]
    sems = rest[8:10]
    acc = rest[10]
    c = lax.axis_index("c")
    s = lax.axis_index("s")
    z0 = s * ZR
    base_e = s * EPG

    def drain(b):
        pltpu.make_async_copy(rows[b], acc.at[idx_d[b]], sems[b]).wait()

    for p in range(2):
        q = c * 2 + p
        pltpu.sync_copy(zeros, acc.at[pl.ds(z0, ZR)])
        plsc.subcore_barrier()

        def pair(i2, carry):
            gcps = []
            for b in range(2):
                e0 = base_e + (i2 * 2 + b) * CB

                @pl.when(i2 > 0)
                def _():
                    drain(b)

                pltpu.sync_copy(srcb.at[pl.ds(q * EPAD + e0, CB)], idx_s[b])
                pltpu.sync_copy(dst1.at[pl.ds(e0, CB)], idx_d[b])
                gcps.append(pltpu.async_copy(hq.at[idx_s[b]], rows[b], semg[b]))
            for b in range(2):
                gcps[b].wait()
                pltpu.async_copy(rows[b], acc.at[idx_d[b]], sems[b], add=True)
            return carry

        lax.fori_loop(0, NPAIR, pair, 0)
        for b in range(2):
            drain(b)
        plsc.subcore_barrier()
        pltpu.sync_copy(acc.at[pl.ds(z0, ZR)], aggq.at[pl.ds(q * NPAD + z0, ZR)])
        plsc.subcore_barrier()


def _sc_deg_body(dst2, onesd, zerosd, deg2, idx_d, ones_v, acc):
    c = lax.axis_index("c")
    s = lax.axis_index("s")
    wid = s * NC + c
    z0 = s * ZR
    pltpu.sync_copy(zerosd, acc.at[pl.ds(z0, ZR)])
    pltpu.sync_copy(onesd, ones_v)
    plsc.subcore_barrier()

    base = wid * RT

    def chunk(i, carry):
        r0 = base + i * CR
        pltpu.sync_copy(dst2.at[pl.ds(r0, CR)], idx_d)
        for j in range(CR):
            pltpu.sync_copy(ones_v, acc.at[idx_d.at[j]], add=True)
        return carry

    lax.fori_loop(0, NCHUNK, chunk, 0)
    plsc.subcore_barrier()
    pltpu.sync_copy(acc.at[pl.ds(z0, ZR)], deg2.at[pl.ds(c * NPAD + z0, ZR)])


@functools.cache
def _build_sc_kernels():
    mesh = plsc.VectorSubcoreMesh(
        core_axis_name="c", subcore_axis_name="s", num_cores=NC, num_subcores=NS
    )
    sc_agg = pl.kernel(
        _sc_agg_body,
        out_type=jax.ShapeDtypeStruct((4 * NPAD, QD), jnp.float32),
        mesh=mesh,
        compiler_params=_sc_params,
        scratch_types=[pltpu.VMEM((CB,), jnp.int32) for _ in range(2)]
        + [pltpu.VMEM((CB,), jnp.int32) for _ in range(2)]
        + [pltpu.VMEM((CB, QD), jnp.float32) for _ in range(2)]
        + [pltpu.SemaphoreType.DMA for _ in range(2)]
        + [pltpu.SemaphoreType.DMA for _ in range(2)]
        + [pltpu.VMEM_SHARED((NPAD, QD), jnp.float32)],
    )
    sc_deg = pl.kernel(
        _sc_deg_body,
        out_type=jax.ShapeDtypeStruct((2 * NPAD, DEGW), jnp.float32),
        mesh=mesh,
        compiler_params=_sc_params,
        scratch_types=[
            pltpu.VMEM((CR, 128), jnp.int32),
            pltpu.VMEM((128, DEGW), jnp.float32),
            pltpu.VMEM_SHARED((NPAD, DEGW), jnp.float32),
        ],
    )
    return sc_agg, sc_deg


B8 = 128
BD = B8 // 2
GRID = Q8 // B8


def _tc_body(h, a, d0, d1, ws, b8, pmat, smat, o_ref, *, last):
    f32 = jnp.float32
    d = d0[0] + d1[0]
    drep = jnp.dot(pmat[...], d, preferred_element_type=f32)
    rid = lax.broadcasted_iota(jnp.int32, (B8, 1), 0)
    dsel = jnp.where(
        rid % 2 == 0,
        jnp.dot(drep, smat[0], preferred_element_type=f32),
        jnp.dot(drep, smat[1], preferred_element_type=f32),
    )
    inv = 1.0 / jnp.maximum(dsel, 1.0)
    z = b8[...]
    for q in range(4):
        z = z + jnp.dot(h[q], ws[q], preferred_element_type=f32)
    for q in range(4):
        z = z + jnp.dot(a[q] * inv, ws[4 + q], preferred_element_type=f32)
    if last:
        parts = []
        for g in range(8):
            zg = z[:, 64 * g : 64 * g + 64]
            n2 = jnp.sum(zg * zg, axis=1, keepdims=True)
            nrm = jnp.maximum(jnp.sqrt(n2), 1e-12)
            parts.append(zg / nrm)
        o_ref[...] = jnp.concatenate(parts, axis=1)
    else:
        z = jnp.maximum(z, 0.0)
        for q in range(4):
            o_ref[q] = jnp.concatenate(
                [z[:, 64 * g + QD * q : 64 * g + QD * q + QD] for g in range(8)],
                axis=1,
            )


def _tc_layer(hT, aggT, degT, WS, b8, pmat, smat, last):
    out_shape = (
        jax.ShapeDtypeStruct((Q8, 8 * D), jnp.float32)
        if last
        else jax.ShapeDtypeStruct((4, Q8, 128), jnp.float32)
    )
    out_spec = (
        pl.BlockSpec((B8, 8 * D), lambda i: (i, 0))
        if last
        else pl.BlockSpec((4, B8, 128), lambda i: (0, i, 0))
    )
    return pl.pallas_call(
        functools.partial(_tc_body, last=last),
        grid=(GRID,),
        in_specs=[
            pl.BlockSpec((4, B8, 128), lambda i: (0, i, 0)),
            pl.BlockSpec((4, B8, 128), lambda i: (0, i, 0)),
            pl.BlockSpec((1, BD, 128), lambda i: (0, i, 0)),
            pl.BlockSpec((1, BD, 128), lambda i: (1, i, 0)),
            pl.BlockSpec((8, 2 * D, 8 * D), lambda i: (0, 0, 0)),
            pl.BlockSpec((1, 8 * D), lambda i: (0, 0)),
            pl.BlockSpec((B8, BD), lambda i: (0, 0)),
            pl.BlockSpec((2, 128, 128), lambda i: (0, 0, 0)),
        ],
        out_specs=out_spec,
        out_shape=out_shape,
    )(hT, aggT, degT, degT, WS, b8, pmat, smat)


def _block_diag8(w):
    z = jnp.zeros((QD, D), jnp.float32)
    rows = []
    for i in range(8):
        cols = [w if j == i else z for j in range(8)]
        rows.append(jnp.concatenate(cols, axis=1))
    return jnp.concatenate(rows, axis=0)


def _pack_weights(W, b):
    WS = jnp.stack(
        [_block_diag8(W[QD * k : QD * k + QD]) for k in range(8)]
    )
    b8 = jnp.tile(b.reshape(1, D), (1, 8))
    return WS, b8


def kernel(x, edge_index, W0, b0, W1, b1):
    src = edge_index[0]
    dst = edge_index[1]
    xp = jnp.pad(x, ((0, NPAD - N), (0, 0)))
    hT0 = jnp.stack(
        [xp[:, QD * q : QD * q + QD].reshape(Q8, 128) for q in range(4)]
    )

    src_p = jnp.concatenate([src, jnp.zeros((EPAD - E,), jnp.int32)])
    dst_p = jnp.concatenate([dst, jnp.full((EPAD - E,), N, jnp.int32)])
    srcb = jnp.concatenate([src_p + q * NPAD for q in range(4)])
    dst2 = dst_p.reshape(ER, 128)

    zeros = jnp.zeros((ZR, QD), jnp.float32)
    zerosd = jnp.zeros((ZR, DEGW), jnp.float32)
    onesd = jnp.ones((128, DEGW), jnp.float32)
    pmat = (
        jnp.arange(B8)[:, None] // 2 == jnp.arange(BD)[None, :]
    ).astype(jnp.float32)
    lane = jnp.arange(128)
    smat = jnp.stack(
        [
            (jnp.arange(128)[:, None] == 64 * m + 8 * (lane[None, :] // 16)).astype(
                jnp.float32
            )
            for m in range(2)
        ]
    )
    WS0, b80 = _pack_weights(W0, b0)
    WS1, b81 = _pack_weights(W1, b1)

    sc_agg, sc_deg = _build_sc_kernels()
    degT = sc_deg(dst2, onesd, zerosd).reshape(2, Q16, 128)

    agg0 = sc_agg(hT0.reshape(4 * NPAD, QD), srcb, dst_p, zeros)
    h1T = _tc_layer(
        hT0, agg0.reshape(4, Q8, 128), degT, WS0, b80, pmat, smat, False
    )

    agg1 = sc_agg(h1T.reshape(4 * NPAD, QD), srcb, dst_p, zeros)
    outp = _tc_layer(
        h1T, agg1.reshape(4, Q8, 128), degT, WS1, b81, pmat, smat, True
    )

    return outp.reshape(NPAD, D)[:N]
